# speedup vs baseline: 1.0000x; 1.0000x over previous
import jax
import jax.numpy as jnp
from jax.experimental import pallas as pl

N = 50000
E = 800000
H = 256
G = 128
EPS = 1e-5


def _gcn(x, W, b, src, dst):
    n = x.shape[0]
    h = x @ W
    loop = jnp.arange(n, dtype=src.dtype)
    s = jnp.concatenate([src, loop])
    d = jnp.concatenate([dst, loop])
    deg = jnp.zeros((n,), h.dtype).at[d].add(1.0)
    dis = jax.lax.rsqrt(deg)
    norm = (dis[s] * dis[d])[:, None]
    out = jnp.zeros_like(h).at[d].add(h[s] * norm)
    return out + b


def _bn(x, g, b):
    m = jnp.mean(x, axis=0)
    v = jnp.var(x, axis=0)
    return (x - m) * jax.lax.rsqrt(v + EPS) * g + b


def _pools(x, batch):
    cnt = jax.ops.segment_sum(jnp.ones((x.shape[0],), x.dtype), batch, num_segments=G)
    sm = jax.ops.segment_sum(x, batch, num_segments=G)
    mean = sm / jnp.maximum(cnt, 1.0)[:, None]
    mx = jax.ops.segment_max(x, batch, num_segments=G)
    mx = jnp.where(cnt[:, None] > 0, mx, 0.0)
    return jnp.concatenate([mean, mx], axis=1)


def _heads_kernel(p_ref, Ws1_ref, bs1_ref, Ws2_ref, bs2_ref,
                  Ha1_ref, ha1_ref, Hb1_ref, hb1_ref,
                  Ha2_ref, ha2_ref, Hb2_ref, hb2_ref,
                  Ha3_ref, ha3_ref, Hb3_ref, hb3_ref,
                  o1_ref, o2_ref, o3_ref):
    p = p_ref[...]
    s = jnp.maximum(p @ Ws1_ref[...] + bs1_ref[...][None, :], 0.0)
    s = jnp.maximum(s @ Ws2_ref[...] + bs2_ref[...][None, :], 0.0)
    for Ha, ha, Hb, hb, o in ((Ha1_ref, ha1_ref, Hb1_ref, hb1_ref, o1_ref),
                              (Ha2_ref, ha2_ref, Hb2_ref, hb2_ref, o2_ref),
                              (Ha3_ref, ha3_ref, Hb3_ref, hb3_ref, o3_ref)):
        t = jnp.maximum(s @ Ha[...] + ha[...][None, :], 0.0)
        o[...] = (t @ Hb[...] + hb[...][None, :])[:, 0]


def kernel(x, edge_index, batch, W1, b1, W2, b2, W3, b3, W4, b4,
           g1, be1, g2, be2, g3, be3, g4, be4,
           Ws1, bs1, Ws2, bs2,
           Ha1, ha1, Hb1, hb1, Ha2, ha2, Hb2, hb2, Ha3, ha3, Hb3, hb3):
    src = edge_index[0]
    dst = edge_index[1]
    h = jax.nn.relu(_bn(_gcn(x, W1, b1, src, dst), g1, be1))
    h = jax.nn.relu(_bn(_gcn(h, W2, b2, src, dst), g2, be2))
    h = jax.nn.relu(_bn(_gcn(h, W3, b3, src, dst), g3, be3))
    h = jax.nn.relu(_bn(_gcn(h, W4, b4, src, dst), g4, be4))
    p = _pools(h, batch)

    o1, o2, o3 = pl.pallas_call(
        _heads_kernel,
        out_shape=(jax.ShapeDtypeStruct((G,), jnp.float32),) * 3,
    )(p, Ws1, bs1, Ws2, bs2, Ha1, ha1, Hb1, hb1, Ha2, ha2, Hb2, hb2, Ha3, ha3, Hb3, hb3)
    return (o1, o2, o3)


# speedup vs baseline: 3.4416x; 3.4416x over previous
import functools

import jax
import jax.numpy as jnp
from jax import lax
from jax.experimental import pallas as pl
from jax.experimental.pallas import tpu as pltpu
from jax.experimental.pallas import tpu_sc as plsc

N = 50000
E = 800000
DIN = 8
H = 256
G = 128
EPS = 1e-5

NPAD = 50688
TPR = NPAD // 16
NS = 8
SW = 32
BLK = 128
EROWS = 6400
EPADV = NPAD - 1
SROWS = (N + BLK - 1) // BLK
GT = G + 8

_mesh = plsc.VectorSubcoreMesh(core_axis_name="c", subcore_axis_name="s",
                               num_cores=2, num_subcores=16)

_i32 = jnp.int32
_f32 = jnp.float32


def _it16():
    return lax.iota(_i32, 16)


def _dot(a, b):
    return jnp.dot(a, b, precision=lax.Precision.HIGHEST)


def _rsqrt(x):
    r = lax.rsqrt(x)
    return r * (1.5 - 0.5 * x * r * r)


@functools.partial(
    pl.kernel,
    out_type=[jax.ShapeDtypeStruct((2, NPAD), _f32)],
    mesh=_mesh,
    scratch_types=[
        pltpu.VMEM_SHARED((NPAD,), _f32),
        pltpu.VMEM((8, BLK), _i32),
        pltpu.VMEM((BLK,), _f32),
        pltpu.VMEM((1056,), _f32),
    ],
    compiler_params=pltpu.CompilerParams(use_tc_tiling_on_sc=False),
)
def _sc_deg(dst_hbm, deg_hbm, deg_acc, dbuf, ones, zbuf):
    cid = lax.axis_index("c")
    sid = lax.axis_index("s")
    tid = cid * 16 + sid

    def _fill(i, _):
        zbuf[pl.ds(16 * i, 16)] = jnp.zeros((16,), _f32)
        return ()
    lax.fori_loop(0, 66, _fill, ())
    for k in range(8):
        ones[pl.ds(16 * k, 16)] = jnp.ones((16,), _f32)
    for q in range(3):
        pltpu.sync_copy(zbuf, deg_acc.at[pl.ds(sid * TPR + q * 1056, 1056)])
    plsc.subcore_barrier()

    start = tid * 200

    def _grp(gi, _):
        r0 = start + gi * 8
        pltpu.sync_copy(dst_hbm.at[pl.ds(r0, 8)], dbuf)
        for j in range(8):
            pltpu.sync_copy(ones, deg_acc.at[dbuf.at[j]], add=True)
        return ()
    lax.fori_loop(0, 25, _grp, ())

    plsc.subcore_barrier()
    pltpu.sync_copy(deg_acc.at[pl.ds(sid * TPR, TPR)],
                    deg_hbm.at[cid, pl.ds(sid * TPR, TPR)])


def _tc_first_body(x_ref, W_ref, d0_ref, d1_ref, mp_ref, dis_ref):
    deg = d0_ref[...] + d1_ref[...] + 1.0
    dis = _rsqrt(deg)
    mp_ref[...] = _dot(x_ref[...], W_ref[...]) * dis
    dis_ref[...] = dis


def _tc_first(xp, W1, deg0, deg1):
    return pl.pallas_call(
        _tc_first_body,
        grid=(99,),
        in_specs=[
            pl.BlockSpec((512, DIN), lambda i: (i, 0)),
            pl.BlockSpec((DIN, H), lambda i: (0, 0)),
            pl.BlockSpec((512, 1), lambda i: (i, 0)),
            pl.BlockSpec((512, 1), lambda i: (i, 0)),
        ],
        out_specs=[
            pl.BlockSpec((512, H), lambda i: (i, 0)),
            pl.BlockSpec((512, 1), lambda i: (i, 0)),
        ],
        out_shape=[
            jax.ShapeDtypeStruct((NPAD, H), _f32),
            jax.ShapeDtypeStruct((NPAD, 1), _f32),
        ],
    )(xp, W1, deg0, deg1)


@functools.partial(
    pl.kernel,
    out_type=[jax.ShapeDtypeStruct((2, NS * NPAD, SW), _f32)],
    mesh=_mesh,
    scratch_types=[
        pltpu.VMEM_SHARED((NPAD, SW), _f32),
        pltpu.VMEM((8, BLK), _i32),
        pltpu.VMEM((8, BLK), _i32),
        pltpu.VMEM((8, BLK), _i32),
        pltpu.VMEM((BLK, SW), _f32),
        pltpu.VMEM((BLK, SW), _f32),
        pltpu.VMEM((396, SW), _f32),
        pltpu.VMEM((BLK,), _i32),
        pltpu.VMEM((BLK,), _i32),
        pltpu.SemaphoreType.DMA,
        pltpu.SemaphoreType.DMA,
        pltpu.SemaphoreType.DMA,
        pltpu.SemaphoreType.DMA,
    ],
    compiler_params=pltpu.CompilerParams(use_tc_tiling_on_sc=False),
)
def _sc_agg(mp_hbm, src_hbm, dst_hbm, accp_hbm,
            acc, sbuf, dbuf, fbuf, rb0, rb1, zb, tsi, tdi,
            sg0, sg1, ss0, ss1):
    cid = lax.axis_index("c")
    sid = lax.axis_index("s")
    tid = cid * 16 + sid
    it16 = _it16()

    def _fill(i, _):
        for k in range(2):
            zb[i, pl.ds(16 * k, 16)] = jnp.zeros((16,), _f32)
        return ()
    lax.fori_loop(0, 396, _fill, ())

    start = tid * 200
    sstart = tid * 12 + jnp.minimum(tid, 7)
    sn = 12 + jnp.where(tid < 7, 1, 0)

    def _slab(s, _):
        for q in range(8):
            pltpu.sync_copy(zb, acc.at[pl.ds(sid * TPR + q * 396, 396)])
        plsc.subcore_barrier()

        def _grp(gi, _):
            r0 = start + gi * 8
            pltpu.sync_copy(src_hbm.at[pl.ds(r0, 8)], sbuf)
            pltpu.sync_copy(dst_hbm.at[pl.ds(r0, 8)], dbuf)
            for j in range(8):
                for v in range(8):
                    sv = sbuf[j, pl.ds(16 * v, 16)]
                    fbuf[j, pl.ds(16 * v, 16)] = sv * NS + s
            gd = [None] * 8
            sd = [None] * 8
            rbs = (rb0, rb1)
            gsem = (sg0, sg1)
            ssem = (ss0, ss1)
            for j in range(8):
                if j >= 2:
                    sd[j - 2].wait()
                gd[j] = pltpu.async_copy(mp_hbm.at[fbuf.at[j]],
                                         rbs[j % 2], gsem[j % 2])
                if j >= 1:
                    gd[j - 1].wait()
                    sd[j - 1] = pltpu.async_copy(
                        rbs[(j - 1) % 2], acc.at[dbuf.at[j - 1]],
                        ssem[(j - 1) % 2], add=True)
            gd[7].wait()
            sd[7] = pltpu.async_copy(rb1, acc.at[dbuf.at[7]], ss1, add=True)
            sd[6].wait()
            sd[7].wait()
            return ()
        lax.fori_loop(0, 25, _grp, ())

        def _self(b, _):
            for v in range(8):
                dv = b * BLK + 16 * v + it16
                tdi[pl.ds(16 * v, 16)] = dv
                tsi[pl.ds(16 * v, 16)] = dv * NS + s
            pltpu.sync_copy(mp_hbm.at[tsi], rb0)
            pltpu.sync_copy(rb0, acc.at[tdi], add=True)
            return ()
        lax.fori_loop(sstart, sstart + sn, _self, ())

        plsc.subcore_barrier()
        pltpu.sync_copy(acc.at[pl.ds(sid * TPR, TPR)],
                        accp_hbm.at[cid, pl.ds(s * NPAD + sid * TPR, TPR)])
        plsc.subcore_barrier()
        return ()
    lax.fori_loop(0, NS, _slab, ())


def _tc_zpass_body(*refs):
    acc_refs = refs[:16]
    dis_ref, b_ref, z_ref, st_ref, st_acc = refs[16:]
    i = pl.program_id(0)
    pieces = []
    for s in range(NS):
        pieces.append(acc_refs[s][0, 0] + acc_refs[NS + s][0, 0])
    a = jnp.concatenate(pieces, axis=1)
    z = a * dis_ref[...] + b_ref[...][None, :]
    z_ref[...] = z

    @pl.when(i == 0)
    def _init():
        c0 = jnp.sum(z, axis=0) * (1.0 / 512.0)
        st_acc[...] = jnp.concatenate(
            [jnp.zeros((512,), _f32), c0])[None, :]
    c = st_acc[0, 512:768]
    rows = jax.lax.broadcasted_iota(_i32, (512, 1), 0) + i * 512
    zc = jnp.where(rows < N, z - c[None, :], 0.0)
    part = jnp.concatenate([jnp.sum(zc, axis=0), jnp.sum(zc * zc, axis=0),
                            jnp.zeros((256,), _f32)])
    st_acc[...] += part[None, :]
    st_ref[...] = st_acc[...]


def _tc_zpass(accp, dis, b):
    accr = accp.reshape(2, NS, NPAD, SW)
    specs = []
    for c in range(2):
        for s in range(NS):
            specs.append(pl.BlockSpec(
                (1, 1, 512, SW), lambda i, c=c, s=s: (c, s, i, 0)))
    specs += [
        pl.BlockSpec((512, 1), lambda i: (i, 0)),
        pl.BlockSpec((H,), lambda i: (0,)),
    ]
    return pl.pallas_call(
        _tc_zpass_body,
        grid=(99,),
        in_specs=specs,
        out_specs=[
            pl.BlockSpec((512, H), lambda i: (i, 0)),
            pl.BlockSpec((1, 768), lambda i: (0, 0)),
        ],
        out_shape=[
            jax.ShapeDtypeStruct((NPAD, H), _f32),
            jax.ShapeDtypeStruct((1, 768), _f32),
        ],
        scratch_shapes=[pltpu.VMEM((1, 768), _f32)],
    )(*([accr] * 16 + [dis, b]))


def _bn_consts(st_ref, g_ref, be_ref):
    st = st_ref[0]
    d = st[:256] * (1.0 / N)
    var = st[256:512] * (1.0 / N) - d * d
    mean = st[512:768] + d
    scale = _rsqrt(var + EPS) * g_ref[...]
    shift = be_ref[...] - mean * scale
    return scale, shift


def _tc_mid_body(st_ref, z_ref, dis_ref, W_ref, g_ref, be_ref, out_ref):
    scale, shift = _bn_consts(st_ref, g_ref, be_ref)
    y = jnp.maximum(z_ref[...] * scale[None, :] + shift[None, :], 0.0)
    out_ref[...] = _dot(y, W_ref[...]) * dis_ref[...]


def _tc_mid(st, z, dis, W, g, be):
    return pl.pallas_call(
        _tc_mid_body,
        grid=(99,),
        in_specs=[
            pl.BlockSpec((1, 768), lambda i: (0, 0)),
            pl.BlockSpec((512, H), lambda i: (i, 0)),
            pl.BlockSpec((512, 1), lambda i: (i, 0)),
            pl.BlockSpec((H, H), lambda i: (0, 0)),
            pl.BlockSpec((H,), lambda i: (0,)),
            pl.BlockSpec((H,), lambda i: (0,)),
        ],
        out_specs=pl.BlockSpec((512, H), lambda i: (i, 0)),
        out_shape=jax.ShapeDtypeStruct((NPAD, H), _f32),
    )(st, z, dis, W, g, be)


def _tc_last_body(st_ref, z_ref, g_ref, be_ref, y_ref):
    scale, shift = _bn_consts(st_ref, g_ref, be_ref)
    y_ref[...] = jnp.maximum(z_ref[...] * scale[None, :] + shift[None, :], 0.0)


def _tc_last(st, z, g, be):
    return pl.pallas_call(
        _tc_last_body,
        grid=(99,),
        in_specs=[
            pl.BlockSpec((1, 768), lambda i: (0, 0)),
            pl.BlockSpec((512, H), lambda i: (i, 0)),
            pl.BlockSpec((H,), lambda i: (0,)),
            pl.BlockSpec((H,), lambda i: (0,)),
        ],
        out_specs=pl.BlockSpec((512, H), lambda i: (i, 0)),
        out_shape=jax.ShapeDtypeStruct((NPAD, H), _f32),
    )(st, z, g, be)


ROWS_PT = 1568


@functools.partial(
    pl.kernel,
    out_type=[
        jax.ShapeDtypeStruct((2, G, H), _f32),
        jax.ShapeDtypeStruct((2, G), _f32),
    ],
    mesh=_mesh,
    scratch_types=[
        pltpu.VMEM_SHARED((GT, H), _f32),
        pltpu.VMEM_SHARED((GT,), _f32),
        pltpu.VMEM((64, H), _f32),
        pltpu.VMEM((64,), _i32),
        pltpu.VMEM((64,), _i32),
        pltpu.VMEM((64,), _f32),
        pltpu.VMEM((GT, H), _f32),
        pltpu.VMEM((144,), _f32),
    ],
    compiler_params=pltpu.CompilerParams(use_tc_tiling_on_sc=False),
)
def _sc_pool(y_hbm, batch_hbm, psum_hbm, pcnt_hbm,
             shsum, shcnt, yb, bb, badj, ones64, zsum, zc):
    cid = lax.axis_index("c")
    sid = lax.axis_index("s")
    tid = cid * 16 + sid
    it16 = _it16()

    for k in range(4):
        ones64[pl.ds(16 * k, 16)] = jnp.ones((16,), _f32)
    for k in range(9):
        zc[pl.ds(16 * k, 16)] = jnp.zeros((16,), _f32)

    def _zr(i, _):
        for k in range(16):
            zsum[i, pl.ds(16 * k, 16)] = jnp.zeros((16,), _f32)
        return ()
    lax.fori_loop(0, GT, _zr, ())

    @pl.when(sid == 0)
    def _initsh():
        pltpu.sync_copy(zsum, shsum)
        pltpu.sync_copy(zc.at[pl.ds(0, GT)], shcnt)
    plsc.subcore_barrier()

    start = tid * ROWS_PT
    end = start + ROWS_PT
    own = sid * 8

    def _block(bi, _):
        g0 = start + bi * 64
        pltpu.sync_copy(y_hbm.at[pl.ds(g0, 64)], yb)
        pltpu.sync_copy(batch_hbm.at[pl.ds(g0, 64)], bb)
        endv = jnp.full((16,), end, _i32)
        for q in range(4):
            rowv = g0 + 16 * q + it16
            bv = bb[pl.ds(16 * q, 16)]
            badj[pl.ds(16 * q, 16)] = jnp.where(rowv < endv, bv, G)
        pltpu.sync_copy(yb, shsum.at[badj], add=True)
        pltpu.sync_copy(ones64, shcnt.at[badj], add=True)
        return ()
    lax.fori_loop(0, 25, _block, ())
    plsc.subcore_barrier()

    pltpu.sync_copy(shsum.at[pl.ds(own, 8)], psum_hbm.at[cid, pl.ds(own, 8)])

    @pl.when(sid == 0)
    def _wcnt():
        pltpu.sync_copy(shcnt.at[pl.ds(0, G)], pcnt_hbm.at[cid])


def _tc_maxpool_body(y_ref, bt_ref, pm_ref, macc):
    i = pl.program_id(0)

    @pl.when(i == 0)
    def _init():
        macc[...] = jnp.zeros((G, H), _f32)
    yv = y_ref[...]
    bt = bt_ref[...]
    rows = []
    for g in range(G):
        m = bt == g
        rows.append(jnp.max(jnp.where(m, yv, 0.0), axis=0, keepdims=True))
    cand = jnp.concatenate(rows, axis=0)
    macc[...] = jnp.maximum(macc[...], cand)
    pm_ref[...] = macc[...]


def _tc_maxpool(y4, batch2):
    return pl.pallas_call(
        _tc_maxpool_body,
        grid=(99,),
        in_specs=[
            pl.BlockSpec((512, H), lambda i: (i, 0)),
            pl.BlockSpec((512, 1), lambda i: (i, 0)),
        ],
        out_specs=pl.BlockSpec((G, H), lambda i: (0, 0)),
        out_shape=jax.ShapeDtypeStruct((G, H), _f32),
        scratch_shapes=[pltpu.VMEM((G, H), _f32)],
    )(y4, batch2)


def _tc_heads_body(ps_ref, pm_ref, pc_ref,
                   Ws1_ref, bs1_ref, Ws2_ref, bs2_ref,
                   Ha1_ref, ha1_ref, Hb1_ref, hb1_ref,
                   Ha2_ref, ha2_ref, Hb2_ref, hb2_ref,
                   Ha3_ref, ha3_ref, Hb3_ref, hb3_ref,
                   o1_ref, o2_ref, o3_ref):
    cnt = pc_ref[0] + pc_ref[1]
    sm = ps_ref[0] + ps_ref[1]
    mx = jnp.where(cnt > 0, pm_ref[...], 0.0)
    cd = jnp.maximum(cnt, 1.0)
    r = 1.0 / cd
    r = r * (2.0 - cd * r)
    mean = sm * r
    p = jnp.concatenate([mean, mx], axis=1)
    s = jnp.maximum(_dot(p, Ws1_ref[...]) + bs1_ref[...][None, :], 0.0)
    s = jnp.maximum(_dot(s, Ws2_ref[...]) + bs2_ref[...][None, :], 0.0)
    for Ha, ha, Hb, hb, o in ((Ha1_ref, ha1_ref, Hb1_ref, hb1_ref, o1_ref),
                              (Ha2_ref, ha2_ref, Hb2_ref, hb2_ref, o2_ref),
                              (Ha3_ref, ha3_ref, Hb3_ref, hb3_ref, o3_ref)):
        t = jnp.maximum(_dot(s, Ha[...]) + ha[...][None, :], 0.0)
        o[...] = (_dot(t, Hb[...]) + hb[...][None, :])[:, 0]


def kernel(x, edge_index, batch, W1, b1, W2, b2, W3, b3, W4, b4,
           g1, be1, g2, be2, g3, be3, g4, be4,
           Ws1, bs1, Ws2, bs2,
           Ha1, ha1, Hb1, hb1, Ha2, ha2, Hb2, hb2, Ha3, ha3, Hb3, hb3):
    npad_e = EROWS * BLK - E
    pad = jnp.full((npad_e,), EPADV, _i32)
    src2 = jnp.concatenate([edge_index[0], pad]).reshape(EROWS, BLK)
    dst2 = jnp.concatenate([edge_index[1], pad]).reshape(EROWS, BLK)
    xp = jnp.zeros((NPAD, DIN), _f32).at[:N].set(x)
    batchp = jnp.full((NPAD,), G, _i32).at[:N].set(batch)

    (deg_p,) = _sc_deg(dst2)
    mp1, dis2 = _tc_first(xp, W1, deg_p[0].reshape(NPAD, 1),
                          deg_p[1].reshape(NPAD, 1))

    def layer(mp, b):
        (accp,) = _sc_agg(mp.reshape(NS * NPAD, SW), src2, dst2)
        return _tc_zpass(accp, dis2, b)

    z1, st1 = layer(mp1, b1)
    mp2 = _tc_mid(st1, z1, dis2, W2, g1, be1)
    z2, st2 = layer(mp2, b2)
    mp3 = _tc_mid(st2, z2, dis2, W3, g2, be2)
    z3, st3 = layer(mp3, b3)
    mp4 = _tc_mid(st3, z3, dis2, W4, g3, be3)
    z4, st4 = layer(mp4, b4)
    y4 = _tc_last(st4, z4, g4, be4)

    psum, pcnt = _sc_pool(y4, batchp)
    pmax = _tc_maxpool(y4, batchp.reshape(NPAD, 1))

    o1, o2, o3 = pl.pallas_call(
        _tc_heads_body,
        out_shape=(jax.ShapeDtypeStruct((G,), _f32),) * 3,
    )(psum, pmax, pcnt.reshape(2, G, 1),
      Ws1, bs1, Ws2, bs2, Ha1, ha1, Hb1, hb1,
      Ha2, ha2, Hb2, hb2, Ha3, ha3, Hb3, hb3)
    return (o1, o2, o3)
